# baseline (speedup 1.0000x reference)
import jax
import jax.numpy as jnp
from jax.experimental import pallas as pl
from jax.experimental.pallas import tpu as pltpu

_NBINS = 64
_SIGMA = 0.05
_EPSV = 1e-6
_LANES = 128
_SUB = 8
_G = (0.5 * 1.4426950408889634) ** 0.5 / _SIGMA


_GRP = 16


def _hist_body(pred_ref, targ_ref, hx_ref, hy_ref):
    base_iota = jax.lax.broadcasted_iota(
        jnp.int32, (_GRP, _SUB, _LANES), 0).astype(jnp.float32)
    ones_row = jnp.ones((1, _LANES), jnp.float32)

    def one_input(ref, out_ref):
        z = ref[0]
        zmin = jnp.min(z)
        zmax = jnp.max(z)
        scale = _G / (zmax - zmin + _EPSV)
        off = zmin * scale

        for g in range(_NBINS // _GRP):
            cgrid = (base_iota + float(g * _GRP)) * (_G / (_NBINS - 1))

            def body(i, acc):
                zb = ref[0, pl.ds(i * _SUB, _SUB), :] * scale - off
                u = zb[None] - cgrid
                v = cgrid - zb[None]
                return acc + jnp.exp2(u * v)

            acc = jax.lax.fori_loop(
                0, 512 // _SUB, body,
                jnp.zeros((_GRP, _SUB, _LANES), jnp.float32))
            r = jnp.sum(acc, axis=1)
            hist_g = jax.lax.dot_general(
                ones_row, r, (((1,), (1,)), ((), ())),
                preferred_element_type=jnp.float32)
            out_ref[0, :, g * _GRP:(g + 1) * _GRP] = hist_g

    one_input(pred_ref, hx_ref)
    one_input(targ_ref, hy_ref)


def _loss_body(hx_ref, hy_ref, o_ref):
    b = hx_ref.shape[0]
    hx = hx_ref[...].reshape(b, _NBINS) + _EPSV
    hy = hy_ref[...].reshape(b, _NBINS) + _EPSV
    ii = jax.lax.broadcasted_iota(jnp.int32, (_NBINS, _NBINS), 0)
    jj = jax.lax.broadcasted_iota(jnp.int32, (_NBINS, _NBINS), 1)
    tri = (ii <= jj).astype(jnp.float32)
    cx = jax.lax.dot_general(hx, tri, (((1,), (0,)), ((), ())),
                             preferred_element_type=jnp.float32)
    cy = jax.lax.dot_general(hy, tri, (((1,), (0,)), ((), ())),
                             preferred_element_type=jnp.float32)
    d = cx / jnp.sum(hx, axis=-1, keepdims=True) \
        - cy / jnp.sum(hy, axis=-1, keepdims=True)
    sq = jnp.sum(d * d, axis=-1, keepdims=True)
    o_ref[...] = jnp.sum(sq, axis=0, keepdims=True) / (b * _NBINS)


def kernel(pred, target):
    b = pred.shape[0]
    n = pred.size // b
    rows = n // _LANES
    p = pred.reshape(b, rows, _LANES)
    t = target.reshape(b, rows, _LANES)
    hx, hy = pl.pallas_call(
        _hist_body,
        grid=(b,),
        in_specs=[pl.BlockSpec((1, rows, _LANES), lambda i: (i, 0, 0))] * 2,
        out_specs=[pl.BlockSpec((1, 1, _NBINS), lambda i: (i, 0, 0))] * 2,
        out_shape=[jax.ShapeDtypeStruct((b, 1, _NBINS), jnp.float32)] * 2,
        compiler_params=pltpu.CompilerParams(
            dimension_semantics=("arbitrary",),
        ),
        name="soft_hist",
    )(p, t)
    out = pl.pallas_call(
        _loss_body,
        out_shape=jax.ShapeDtypeStruct((1, 1), jnp.float32),
        name="cdf_mse",
    )(hx, hy)
    return out.reshape(())


# speedup vs baseline: 1.9317x; 1.9317x over previous
import jax
import jax.numpy as jnp
from jax.experimental import pallas as pl
from jax.experimental.pallas import tpu as pltpu

_NBINS = 64
_SIGMA = 0.05
_EPSV = 1e-6
_LANES = 128
_SUB = 8
_G = (0.5 * 1.4426950408889634) ** 0.5 / _SIGMA


_GRP = 16
_NGRP = _NBINS // _GRP
_RCHUNK = 4
_DELTA = _G / (_NBINS - 1)


def _hist_body(pred_ref, targ_ref, hx_ref, hy_ref, accx_ref, accy_ref):
    cgbase = jax.lax.broadcasted_iota(
        jnp.int32, (_GRP, _SUB, _LANES), 0).astype(jnp.float32) * _DELTA
    ones_row = jnp.ones((1, _LANES), jnp.float32)

    def one_input(ref, acc_ref, out_ref):
        z = ref[0]
        zmin = jnp.min(z)
        zmax = jnp.max(z)
        scale = _G / (zmax - zmin + _EPSV)
        off = zmin * scale
        acc_ref[...] = jnp.zeros((_NBINS, _SUB, _LANES), jnp.float32)

        def body(i, carry):
            r0 = i * (_RCHUNK * _SUB)
            zbs = [ref[0, pl.ds(r0 + r * _SUB, _SUB), :] * scale - off
                   for r in range(_RCHUNK)]
            for g in range(_NGRP):
                a = acc_ref[g * _GRP:(g + 1) * _GRP]
                for r in range(_RCHUNK):
                    t = zbs[r] - (g * _GRP * _DELTA)
                    u = t[None] - cgbase
                    v = cgbase - t[None]
                    a = a + jnp.exp2(u * v)
                acc_ref[g * _GRP:(g + 1) * _GRP] = a
            return carry

        jax.lax.fori_loop(0, 512 // (_RCHUNK * _SUB), body, 0)
        r2 = jnp.sum(acc_ref[...], axis=1)
        out_ref[0] = jax.lax.dot_general(
            ones_row, r2, (((1,), (1,)), ((), ())),
            preferred_element_type=jnp.float32)

    one_input(pred_ref, accx_ref, hx_ref)
    one_input(targ_ref, accy_ref, hy_ref)


def _loss_body(hx_ref, hy_ref, o_ref):
    b = hx_ref.shape[0]
    hx = hx_ref[...].reshape(b, _NBINS) + _EPSV
    hy = hy_ref[...].reshape(b, _NBINS) + _EPSV
    ii = jax.lax.broadcasted_iota(jnp.int32, (_NBINS, _NBINS), 0)
    jj = jax.lax.broadcasted_iota(jnp.int32, (_NBINS, _NBINS), 1)
    tri = (ii <= jj).astype(jnp.float32)
    cx = jax.lax.dot_general(hx, tri, (((1,), (0,)), ((), ())),
                             preferred_element_type=jnp.float32)
    cy = jax.lax.dot_general(hy, tri, (((1,), (0,)), ((), ())),
                             preferred_element_type=jnp.float32)
    d = cx / jnp.sum(hx, axis=-1, keepdims=True) \
        - cy / jnp.sum(hy, axis=-1, keepdims=True)
    sq = jnp.sum(d * d, axis=-1, keepdims=True)
    o_ref[...] = jnp.sum(sq, axis=0, keepdims=True) / (b * _NBINS)


def kernel(pred, target):
    b = pred.shape[0]
    n = pred.size // b
    rows = n // _LANES
    p = pred.reshape(b, rows, _LANES)
    t = target.reshape(b, rows, _LANES)
    hx, hy = pl.pallas_call(
        _hist_body,
        grid=(b,),
        in_specs=[pl.BlockSpec((1, rows, _LANES), lambda i: (i, 0, 0))] * 2,
        out_specs=[pl.BlockSpec((1, 1, _NBINS), lambda i: (i, 0, 0))] * 2,
        out_shape=[jax.ShapeDtypeStruct((b, 1, _NBINS), jnp.float32)] * 2,
        scratch_shapes=[
            pltpu.VMEM((_NBINS, _SUB, _LANES), jnp.float32),
            pltpu.VMEM((_NBINS, _SUB, _LANES), jnp.float32),
        ],
        compiler_params=pltpu.CompilerParams(
            dimension_semantics=("arbitrary",),
        ),
        name="soft_hist",
    )(p, t)
    out = pl.pallas_call(
        _loss_body,
        out_shape=jax.ShapeDtypeStruct((1, 1), jnp.float32),
        name="cdf_mse",
    )(hx, hy)
    return out.reshape(())
